# baseline (speedup 1.0000x reference)
import functools

import jax
import jax.numpy as jnp
from jax import lax
from jax.experimental import pallas as pl
from jax.experimental.pallas import tpu as pltpu
from jax.experimental.pallas import tpu_sc as plsc

NSEG = 10000
D = 128
NC = 2
NS = 16
NW = NC * NS
B = 128
NBLK = 80
NBUF = 2
G = 8
NGRP = NBLK // G
NE_PAD = NW * NBLK * B
CHUNK = B
NCHUNK = NSEG // CHUNK
ZTAIL = NSEG - NCHUNK * CHUNK
CHUNKS_PER_SUB = -(-NCHUNK // NS)


def _sc_body(h_hbm, src_hbm, dst_hbm, norm_hbm, out_hbm,
             src_v, dst_r, norm_r, rows_v, shared_acc, gsem, ssem, isem):
    c = lax.axis_index("c")
    s = lax.axis_index("s")
    wid = s * NC + c

    pltpu.sync_copy(src_hbm.at[wid], src_v)
    for g0 in range(2):
        pltpu.async_copy(dst_hbm.at[wid, g0], dst_r.at[g0], isem.at[g0])
        pltpu.async_copy(norm_hbm.at[wid, g0], norm_r.at[g0], isem.at[g0])

    zero16 = jnp.zeros((16,), jnp.float32)

    def _zero_row(r, carry):
        for k in range(D // 16):
            rows_v[0, r, pl.ds(16 * k, 16)] = zero16
        return carry

    lax.fori_loop(0, B, _zero_row, 0)

    for t in range(CHUNKS_PER_SUB):
        chunk = s * CHUNKS_PER_SUB + t

        @pl.when(chunk < NCHUNK)
        def _():
            off = pl.multiple_of(chunk * CHUNK, 8)
            pltpu.sync_copy(rows_v.at[0], shared_acc.at[pl.ds(off, CHUNK)])

    @pl.when(s == NS - 1)
    def _():
        pltpu.sync_copy(rows_v.at[0, pl.ds(0, ZTAIL)],
                        shared_acc.at[pl.ds(NCHUNK * CHUNK, ZTAIL)])

    pltpu.async_copy(h_hbm.at[src_v.at[pl.ds(0, B)]], rows_v.at[0], gsem.at[0])
    plsc.subcore_barrier()

    def _scale(b, slot, pos, half):
        def _group(g, c2):
            nvec = norm_r[slot, pos, pl.ds(g * 16, 16)]
            for l in range(16):
                nb = nvec[l]
                e = g * 16 + l
                for k in range(D // 16):
                    sl = pl.ds(16 * k, 16)
                    rows_v[b, e, sl] = rows_v[b, e, sl] * nb
            return c2

        h = B // 32
        lax.fori_loop(half * h, (half + 1) * h, _group, 0, unroll=2)

    def _outer(i, carry):
        for b in range(NBUF):
            j = i * NBUF + b
            bp = (b + NBUF - 1) % NBUF
            grp = j // G
            slot = lax.rem(grp, 2)
            pos = lax.rem(j, G)

            if b == 0:
                @pl.when(pos == 0)
                def _():
                    pltpu.make_async_copy(dst_hbm.at[wid, grp], dst_r.at[slot],
                                          isem.at[slot]).wait()
                    pltpu.make_async_copy(norm_hbm.at[wid, grp],
                                          norm_r.at[slot],
                                          isem.at[slot]).wait()

            pltpu.make_async_copy(h_hbm.at[src_v.at[pl.ds(j * B, B)]],
                                  rows_v.at[b], gsem.at[b]).wait()
            _scale(b, slot, pos, 0)

            @pl.when(j >= 1)
            def _():
                pltpu.make_async_copy(rows_v.at[bp],
                                      shared_acc.at[dst_r.at[0, 0]],
                                      ssem.at[bp]).wait()

            @pl.when(j + 1 < NBLK)
            def _():
                pltpu.async_copy(h_hbm.at[src_v.at[pl.ds((j + 1) * B, B)]],
                                 rows_v.at[bp], gsem.at[bp])

            _scale(b, slot, pos, 1)
            pltpu.async_copy(rows_v.at[b], shared_acc.at[dst_r.at[slot, pos]],
                             ssem.at[b], add=True)

            if b == 0:
                @pl.when((pos == 0) & (grp + 1 < NGRP))
                def _():
                    nslot = lax.rem(grp + 1, 2)
                    pltpu.async_copy(dst_hbm.at[wid, grp + 1], dst_r.at[nslot],
                                     isem.at[nslot])
                    pltpu.async_copy(norm_hbm.at[wid, grp + 1],
                                     norm_r.at[nslot], isem.at[nslot])
        return carry

    lax.fori_loop(0, NBLK // NBUF, _outer, 0)
    lastb = (NBLK - 1) % NBUF
    pltpu.make_async_copy(rows_v.at[lastb], shared_acc.at[dst_r.at[0, 0]],
                          ssem.at[lastb]).wait()
    plsc.subcore_barrier()

    for t in range(CHUNKS_PER_SUB):
        chunk = s * CHUNKS_PER_SUB + t

        @pl.when(chunk < NCHUNK)
        def _():
            off = pl.multiple_of(chunk * CHUNK, 8)
            pltpu.sync_copy(shared_acc.at[pl.ds(off, CHUNK)],
                            out_hbm.at[c, pl.ds(off, CHUNK)])

    @pl.when(s == NS - 1)
    def _():
        pltpu.sync_copy(shared_acc.at[pl.ds(NCHUNK * CHUNK, ZTAIL)],
                        out_hbm.at[c, pl.ds(NCHUNK * CHUNK, ZTAIL)])


_sc_scatter = functools.partial(
    pl.kernel,
    out_type=jax.ShapeDtypeStruct((NC, NSEG, D), jnp.float32),
    mesh=plsc.VectorSubcoreMesh(core_axis_name="c", subcore_axis_name="s"),
    scratch_types=[
        pltpu.VMEM((NBLK * B,), jnp.int32),
        pltpu.VMEM((2, G, B), jnp.int32),
        pltpu.VMEM((2, G, B), jnp.float32),
        pltpu.VMEM((NBUF, B, D), jnp.float32),
        pltpu.VMEM_SHARED((NSEG, D), jnp.float32),
        pltpu.SemaphoreType.DMA((NBUF,)),
        pltpu.SemaphoreType.DMA((NBUF,)),
        pltpu.SemaphoreType.DMA((2,)),
    ],
)(_sc_body)


def _tc_in_body(x_ref, w_ref, b_ref, o_ref):
    o_ref[...] = jnp.maximum(
        jnp.dot(x_ref[...], w_ref[...], preferred_element_type=jnp.float32)
        + b_ref[...], 0.0)


def _tc_mid_body(p_ref, wd_ref, bd_ref, we_ref, be_ref, o_ref):
    agg = p_ref[0] + p_ref[1]
    t = jnp.maximum(
        jnp.dot(agg, wd_ref[...], preferred_element_type=jnp.float32)
        + bd_ref[...], 0.0)
    o_ref[...] = jnp.maximum(
        jnp.dot(t, we_ref[...], preferred_element_type=jnp.float32)
        + be_ref[...], 0.0)


def _tc_out_body(p_ref, wd_ref, bd_ref, wc_ref, bc_ref, o_ref):
    agg = p_ref[0] + p_ref[1]
    t = jnp.maximum(
        jnp.dot(agg, wd_ref[...], preferred_element_type=jnp.float32)
        + bd_ref[...], 0.0)
    o_ref[...] = (
        jnp.dot(t, wc_ref[...], preferred_element_type=jnp.float32)
        + bc_ref[...])


def _tc_call(body, out_cols, *args):
    return pl.pallas_call(
        body,
        out_shape=jax.ShapeDtypeStruct((NSEG, out_cols), jnp.float32),
    )(*args)


def kernel(x, edge_index, norm,
           W_v2e_enc, b_v2e_enc, W_v2e_dec, b_v2e_dec,
           W_e2v_enc, b_e2v_enc, W_e2v_dec, b_e2v_dec,
           W_cls, b_cls):
    n_e = edge_index.shape[1]
    pad = NE_PAD - n_e

    cidx = jnp.min(edge_index[1])
    e_v = edge_index[0]
    e_he = edge_index[1] - cidx

    zpad_i = jnp.arange(pad, dtype=jnp.int32) % NSEG
    zpad_f = jnp.zeros((pad,), jnp.float32)
    src1 = jnp.concatenate([e_v, zpad_i]).reshape(NW, NBLK * B)
    dst1 = jnp.concatenate([e_he, zpad_i]).reshape(NW, NGRP, G, B)
    src2 = jnp.concatenate([e_he, zpad_i]).reshape(NW, NBLK * B)
    dst2 = jnp.concatenate([e_v, zpad_i]).reshape(NW, NGRP, G, B)
    nrm = jnp.concatenate([norm, zpad_f]).reshape(NW, NGRP, G, B)

    wc = jnp.zeros((D, D), jnp.float32).at[:, :W_cls.shape[1]].set(W_cls)
    bc = jnp.zeros((D,), jnp.float32).at[:W_cls.shape[1]].set(b_cls)

    b1 = b_v2e_enc.reshape(1, D)
    b2 = b_v2e_dec.reshape(1, D)
    b3 = b_e2v_enc.reshape(1, D)
    b4 = b_e2v_dec.reshape(1, D)
    bc = bc.reshape(1, D)

    h1 = _tc_call(_tc_in_body, D, x, W_v2e_enc, b1)
    p1 = _sc_scatter(h1, src1, dst1, nrm)
    h3 = _tc_call(_tc_mid_body, D, p1, W_v2e_dec, b2, W_e2v_enc, b3)
    p2 = _sc_scatter(h3, src2, dst2, nrm)
    out = _tc_call(_tc_out_body, D, p2, W_e2v_dec, b4, wc, bc)
    return out[:, :W_cls.shape[1]]


# speedup vs baseline: 1.0082x; 1.0082x over previous
import functools

import jax
import jax.numpy as jnp
from jax import lax
from jax.experimental import pallas as pl
from jax.experimental.pallas import tpu as pltpu
from jax.experimental.pallas import tpu_sc as plsc

NSEG = 10000
D = 128
NC = 2
NS = 16
NW = NC * NS
B = 128
NBLK = 80
NBUF = 2
G = 8
NGRP = NBLK // G
NE_PAD = NW * NBLK * B
CHUNK = B
NCHUNK = NSEG // CHUNK
ZTAIL = NSEG - NCHUNK * CHUNK
CHUNKS_PER_SUB = -(-NCHUNK // NS)


def _sc_body(h_hbm, src_hbm, dst_hbm, norm_hbm, out_hbm,
             src_v, dst_r, norm_r, rows_v, shared_acc, gsem, ssem, isem):
    c = lax.axis_index("c")
    s = lax.axis_index("s")
    wid = s * NC + c

    pltpu.sync_copy(src_hbm.at[wid], src_v)
    for g0 in range(2):
        pltpu.async_copy(dst_hbm.at[wid, g0], dst_r.at[g0], isem.at[g0])
        pltpu.async_copy(norm_hbm.at[wid, g0], norm_r.at[g0], isem.at[g0])

    pltpu.async_copy(h_hbm.at[src_v.at[pl.ds(0, B)]], rows_v.at[0],
                     gsem.at[0])

    zero16 = jnp.zeros((16,), jnp.float32)

    def _zero_row(r, carry):
        for k in range(D // 16):
            rows_v[1, r, pl.ds(16 * k, 16)] = zero16
        return carry

    lax.fori_loop(0, B, _zero_row, 0)

    for t in range(CHUNKS_PER_SUB):
        chunk = s * CHUNKS_PER_SUB + t

        @pl.when(chunk < NCHUNK)
        def _():
            off = pl.multiple_of(chunk * CHUNK, 8)
            pltpu.async_copy(rows_v.at[1], shared_acc.at[pl.ds(off, CHUNK)],
                             ssem.at[1])

    @pl.when(s == NS - 1)
    def _():
        pltpu.async_copy(rows_v.at[1, pl.ds(0, ZTAIL)],
                         shared_acc.at[pl.ds(NCHUNK * CHUNK, ZTAIL)],
                         ssem.at[0])
        pltpu.make_async_copy(rows_v.at[1, pl.ds(0, ZTAIL)],
                              shared_acc.at[pl.ds(NCHUNK * CHUNK, ZTAIL)],
                              ssem.at[0]).wait()

    for t in range(CHUNKS_PER_SUB):
        chunk = s * CHUNKS_PER_SUB + t

        @pl.when(chunk < NCHUNK)
        def _():
            pltpu.make_async_copy(rows_v.at[1],
                                  shared_acc.at[pl.ds(0, CHUNK)],
                                  ssem.at[1]).wait()
    plsc.subcore_barrier()

    def _scale(b, slot, pos, half):
        def _group(g, c2):
            nvec = norm_r[slot, pos, pl.ds(g * 16, 16)]
            for l in range(16):
                nb = nvec[l]
                e = g * 16 + l
                for k in range(D // 16):
                    sl = pl.ds(16 * k, 16)
                    rows_v[b, e, sl] = rows_v[b, e, sl] * nb
            return c2

        h = B // 32
        lax.fori_loop(half * h, (half + 1) * h, _group, 0)

    def _outer(i, carry):
        for b in range(NBUF):
            j = i * NBUF + b
            bp = (b + NBUF - 1) % NBUF
            grp = j // G
            slot = lax.rem(grp, 2)
            pos = lax.rem(j, G)

            if b == 0:
                @pl.when(pos == 0)
                def _():
                    pltpu.make_async_copy(dst_hbm.at[wid, grp], dst_r.at[slot],
                                          isem.at[slot]).wait()
                    pltpu.make_async_copy(norm_hbm.at[wid, grp],
                                          norm_r.at[slot],
                                          isem.at[slot]).wait()

            pltpu.make_async_copy(h_hbm.at[src_v.at[pl.ds(j * B, B)]],
                                  rows_v.at[b], gsem.at[b]).wait()
            _scale(b, slot, pos, 0)

            @pl.when(j >= 1)
            def _():
                pltpu.make_async_copy(rows_v.at[bp],
                                      shared_acc.at[dst_r.at[0, 0]],
                                      ssem.at[bp]).wait()

            @pl.when(j + NBUF - 1 < NBLK)
            def _():
                pltpu.async_copy(
                    h_hbm.at[src_v.at[pl.ds((j + NBUF - 1) * B, B)]],
                    rows_v.at[bp], gsem.at[bp])

            _scale(b, slot, pos, 1)
            pltpu.async_copy(rows_v.at[b], shared_acc.at[dst_r.at[slot, pos]],
                             ssem.at[b], add=True)

            if b == 0:
                @pl.when((pos == 0) & (grp + 1 < NGRP))
                def _():
                    nslot = lax.rem(grp + 1, 2)
                    pltpu.async_copy(dst_hbm.at[wid, grp + 1], dst_r.at[nslot],
                                     isem.at[nslot])
                    pltpu.async_copy(norm_hbm.at[wid, grp + 1],
                                     norm_r.at[nslot], isem.at[nslot])
        return carry

    lax.fori_loop(0, NBLK // NBUF, _outer, 0)
    lastb = (NBLK - 1) % NBUF
    pltpu.make_async_copy(rows_v.at[lastb], shared_acc.at[dst_r.at[0, 0]],
                          ssem.at[lastb]).wait()
    plsc.subcore_barrier()

    for t in range(CHUNKS_PER_SUB):
        chunk = s * CHUNKS_PER_SUB + t

        @pl.when(chunk < NCHUNK)
        def _():
            off = pl.multiple_of(chunk * CHUNK, 8)
            pltpu.async_copy(shared_acc.at[pl.ds(off, CHUNK)],
                             out_hbm.at[c, pl.ds(off, CHUNK)], gsem.at[0])

    @pl.when(s == NS - 1)
    def _():
        pltpu.sync_copy(shared_acc.at[pl.ds(NCHUNK * CHUNK, ZTAIL)],
                        out_hbm.at[c, pl.ds(NCHUNK * CHUNK, ZTAIL)])

    for t in range(CHUNKS_PER_SUB):
        chunk = s * CHUNKS_PER_SUB + t

        @pl.when(chunk < NCHUNK)
        def _():
            pltpu.make_async_copy(shared_acc.at[pl.ds(0, CHUNK)],
                                  out_hbm.at[c, pl.ds(0, CHUNK)],
                                  gsem.at[0]).wait()


_sc_scatter = functools.partial(
    pl.kernel,
    out_type=jax.ShapeDtypeStruct((NC, NSEG, D), jnp.float32),
    mesh=plsc.VectorSubcoreMesh(core_axis_name="c", subcore_axis_name="s"),
    scratch_types=[
        pltpu.VMEM((NBLK * B,), jnp.int32),
        pltpu.VMEM((2, G, B), jnp.int32),
        pltpu.VMEM((2, G, B), jnp.float32),
        pltpu.VMEM((NBUF, B, D), jnp.float32),
        pltpu.VMEM_SHARED((NSEG, D), jnp.float32),
        pltpu.SemaphoreType.DMA((NBUF,)),
        pltpu.SemaphoreType.DMA((NBUF,)),
        pltpu.SemaphoreType.DMA((2,)),
    ],
)(_sc_body)


def _tc_in_body(x_ref, w_ref, b_ref, o_ref):
    o_ref[...] = jnp.maximum(
        jnp.dot(x_ref[...], w_ref[...], preferred_element_type=jnp.float32)
        + b_ref[...], 0.0)


def _tc_mid_body(p_ref, wd_ref, bd_ref, we_ref, be_ref, o_ref):
    agg = p_ref[0] + p_ref[1]
    t = jnp.maximum(
        jnp.dot(agg, wd_ref[...], preferred_element_type=jnp.float32)
        + bd_ref[...], 0.0)
    o_ref[...] = jnp.maximum(
        jnp.dot(t, we_ref[...], preferred_element_type=jnp.float32)
        + be_ref[...], 0.0)


def _tc_out_body(p_ref, wd_ref, bd_ref, wc_ref, bc_ref, o_ref):
    agg = p_ref[0] + p_ref[1]
    t = jnp.maximum(
        jnp.dot(agg, wd_ref[...], preferred_element_type=jnp.float32)
        + bd_ref[...], 0.0)
    o_ref[...] = (
        jnp.dot(t, wc_ref[...], preferred_element_type=jnp.float32)
        + bc_ref[...])


def _tc_call(body, out_cols, *args):
    return pl.pallas_call(
        body,
        out_shape=jax.ShapeDtypeStruct((NSEG, out_cols), jnp.float32),
    )(*args)


def kernel(x, edge_index, norm,
           W_v2e_enc, b_v2e_enc, W_v2e_dec, b_v2e_dec,
           W_e2v_enc, b_e2v_enc, W_e2v_dec, b_e2v_dec,
           W_cls, b_cls):
    n_e = edge_index.shape[1]
    pad = NE_PAD - n_e

    cidx = jnp.min(edge_index[1])
    e_v = edge_index[0]
    e_he = edge_index[1] - cidx

    zpad_i = jnp.arange(pad, dtype=jnp.int32) % NSEG
    zpad_f = jnp.zeros((pad,), jnp.float32)
    src1 = jnp.concatenate([e_v, zpad_i]).reshape(NW, NBLK * B)
    dst1 = jnp.concatenate([e_he, zpad_i]).reshape(NW, NGRP, G, B)
    src2 = jnp.concatenate([e_he, zpad_i]).reshape(NW, NBLK * B)
    dst2 = jnp.concatenate([e_v, zpad_i]).reshape(NW, NGRP, G, B)
    nrm = jnp.concatenate([norm, zpad_f]).reshape(NW, NGRP, G, B)

    wc = jnp.zeros((D, D), jnp.float32).at[:, :W_cls.shape[1]].set(W_cls)
    bc = jnp.zeros((D,), jnp.float32).at[:W_cls.shape[1]].set(b_cls)

    b1 = b_v2e_enc.reshape(1, D)
    b2 = b_v2e_dec.reshape(1, D)
    b3 = b_e2v_enc.reshape(1, D)
    b4 = b_e2v_dec.reshape(1, D)
    bc = bc.reshape(1, D)

    h1 = _tc_call(_tc_in_body, D, x, W_v2e_enc, b1)
    p1 = _sc_scatter(h1, src1, dst1, nrm)
    h3 = _tc_call(_tc_mid_body, D, p1, W_v2e_dec, b2, W_e2v_enc, b3)
    p2 = _sc_scatter(h3, src2, dst2, nrm)
    out = _tc_call(_tc_out_body, D, p2, W_e2v_dec, b4, wc, bc)
    return out[:, :W_cls.shape[1]]
